# Initial kernel scaffold; baseline (speedup 1.0000x reference)
import jax
import jax.numpy as jnp
from jax.experimental import pallas as pl


def kernel(x, edge_index, W0, b0, W1, b1):
    raise NotImplementedError("write your pallas kernel here")


# speedup vs baseline: 23.3668x; 23.3668x over previous
import functools

import jax
import jax.numpy as jnp
from jax import lax
from jax.experimental import pallas as pl
from jax.experimental.pallas import tpu as pltpu
from jax.experimental.pallas import tpu_sc as plsc

N = 10000
D = 128
E = 320000
NC = 2
NS = 16
NW = NC * NS
EPW = E // NW
K = 125
C = EPW // K
IB = 8
NROWS = 10240
RPT = NROWS // NS
DPT = NROWS // NS
ZR = 64

_mesh = plsc.VectorSubcoreMesh(core_axis_name="c", subcore_axis_name="s")


@functools.partial(
    pl.kernel,
    out_type=jax.ShapeDtypeStruct((NC * NROWS,), jnp.float32),
    mesh=_mesh,
    scratch_types=dict(
        deg=pltpu.VMEM_SHARED((NROWS,), jnp.float32),
        dst_v=pltpu.VMEM((C, K), jnp.int32),
        ones_v=pltpu.VMEM((K,), jnp.float32),
        zbuf=pltpu.VMEM((DPT,), jnp.float32),
    ),
)
def _deg_kernel(dsts_hbm, out_hbm, *, deg, dst_v, ones_v, zbuf):
    cid = lax.axis_index("c")
    sid = lax.axis_index("s")
    wid = sid * NC + cid

    def zbody(i, _):
        zbuf[pl.ds(i * 16, 16)] = jnp.zeros((16,), jnp.float32)
        return 0

    lax.fori_loop(0, DPT // 16, zbody, 0)
    pltpu.sync_copy(zbuf, deg.at[pl.ds(sid * DPT, DPT)])

    ones_offs = list(range(0, K - 15, 16))
    if ones_offs[-1] + 16 < K:
        ones_offs.append(K - 16)
    for off in ones_offs:
        ones_v[pl.ds(off, 16)] = jnp.ones((16,), jnp.float32)

    pltpu.sync_copy(dsts_hbm.at[wid], dst_v)
    plsc.subcore_barrier()

    def chunk(j, _):
        pltpu.sync_copy(ones_v, deg.at[dst_v.at[j]], add=True)
        return 0

    lax.fori_loop(0, C, chunk, 0)
    plsc.subcore_barrier()

    pltpu.sync_copy(deg.at[pl.ds(sid * DPT, DPT)],
                    out_hbm.at[pl.ds(cid * NROWS + sid * DPT, DPT)])


@functools.partial(
    pl.kernel,
    out_type=jax.ShapeDtypeStruct((NC, NROWS, D), jnp.float32),
    mesh=_mesh,
    scratch_types=dict(
        acc=pltpu.VMEM_SHARED((NROWS, D), jnp.float32),
        src_v=pltpu.VMEM((IB, K), jnp.int32),
        dst_v=pltpu.VMEM((IB, K), jnp.int32),
        buf=pltpu.VMEM((K, D), jnp.float32),
        zbuf=pltpu.VMEM((ZR, D), jnp.float32),
    ),
)
def _agg_kernel(g_hbm, srcs_hbm, dsts_hbm, out_hbm, *, acc, src_v, dst_v, buf, zbuf):
    cid = lax.axis_index("c")
    sid = lax.axis_index("s")
    wid = sid * NC + cid

    def zbody(i, _):
        for c in range(D // 16):
            zbuf[i, pl.ds(c * 16, 16)] = jnp.zeros((16,), jnp.float32)
        return 0

    lax.fori_loop(0, ZR, zbody, 0)
    for t in range(RPT // ZR):
        pltpu.sync_copy(zbuf, acc.at[pl.ds(sid * RPT + t * ZR, ZR)])
    plsc.subcore_barrier()

    def block(b, _):
        pltpu.sync_copy(srcs_hbm.at[wid, pl.ds(b * IB, IB)], src_v)
        pltpu.sync_copy(dsts_hbm.at[wid, pl.ds(b * IB, IB)], dst_v)

        def chunk(t, _):
            pltpu.sync_copy(g_hbm.at[src_v.at[t]], buf)
            pltpu.sync_copy(buf, acc.at[dst_v.at[t]], add=True)
            return 0

        lax.fori_loop(0, IB, chunk, 0)
        return 0

    lax.fori_loop(0, C // IB, block, 0)
    plsc.subcore_barrier()

    pltpu.sync_copy(acc.at[pl.ds(sid * RPT, RPT)],
                    out_hbm.at[cid, pl.ds(sid * RPT, RPT)])


def _tc1_body(degp_ref, x_ref, w_ref, dinv_ref, g_ref):
    deg = degp_ref[...].sum(axis=0) + 1.0
    dinv = lax.rsqrt(deg)[:, None]
    dinv_ref[...] = dinv
    h = jnp.dot(x_ref[...], w_ref[...], preferred_element_type=jnp.float32)
    g_ref[...] = h * dinv[:N]


def _tc2_body(accp_ref, g_ref, dinv_ref, b_ref, w_ref, gnext_ref):
    dv = dinv_ref[:N]
    acc = accp_ref[0, :N] + accp_ref[1, :N]
    out = dv * (acc + g_ref[...]) + b_ref[...]
    h = jnp.maximum(out, 0.0)
    gnext_ref[...] = dv * jnp.dot(h, w_ref[...], preferred_element_type=jnp.float32)


def _tc3_body(accp_ref, g_ref, dinv_ref, b_ref, out_ref):
    dv = dinv_ref[:N]
    acc = accp_ref[0, :N] + accp_ref[1, :N]
    out_ref[...] = dv * (acc + g_ref[...]) + b_ref[...]


def kernel(x, edge_index, W0, b0, W1, b1):
    src3 = edge_index[0].reshape(NW, C, K)
    dst3 = edge_index[1].reshape(NW, C, K)

    deg_p = _deg_kernel(dst3).reshape(NC, NROWS)

    dinv, g0 = pl.pallas_call(
        _tc1_body,
        out_shape=(
            jax.ShapeDtypeStruct((NROWS, 1), jnp.float32),
            jax.ShapeDtypeStruct((N, D), jnp.float32),
        ),
    )(deg_p, x, W0)

    acc0 = _agg_kernel(g0, src3, dst3)

    g1 = pl.pallas_call(
        _tc2_body,
        out_shape=jax.ShapeDtypeStruct((N, D), jnp.float32),
    )(acc0, g0, dinv, b0.reshape(1, D), W1)

    acc1 = _agg_kernel(g1, src3, dst3)

    out = pl.pallas_call(
        _tc3_body,
        out_shape=jax.ShapeDtypeStruct((N, D), jnp.float32),
    )(acc1, g1, dinv, b1.reshape(1, D))

    return out
